# baseline (speedup 1.0000x reference)
import functools

import jax
import jax.numpy as jnp
from jax import lax
from jax.experimental import pallas as pl
from jax.experimental.pallas import tpu as pltpu
from jax.experimental.pallas import tpu_sc as plsc

ENT_COUNT = 1000000
WORD_VOCAB = 100000
VEC = 64
ROWS = 204800
JPB = 200
NC, NS, L = 2, 16, 16
NW = NC * NS
RPW = ROWS // NW
EPW = 32
NBLK = EPW // L
RPB = L * JPB
PIECE = 128
PPW = RPW // PIECE
PPB = PPW // NBLK
JPP = PIECE // L
PAIR = 2 * VEC


def _rsqrt(x):
    i = plsc.bitcast(x, jnp.int32)
    y = plsc.bitcast(jnp.int32(0x5F3759DF) - lax.shift_right_logical(i, 1),
                     jnp.float32)
    for _ in range(3):
        y = y * (1.5 - 0.5 * x * y * y)
    return y


@functools.partial(
    pl.kernel,
    out_type=jax.ShapeDtypeStruct((ROWS,), jnp.float32),
    mesh=plsc.VectorSubcoreMesh(core_axis_name="c", subcore_axis_name="s",
                                num_cores=NC, num_subcores=NS),
    scratch_types=[
        pltpu.VMEM((RPW,), jnp.int32),
        pltpu.VMEM((PPW, PIECE), jnp.int32),
        pltpu.VMEM((RPW,), jnp.int32),
        pltpu.VMEM((NBLK, L), jnp.int32),
        pltpu.VMEM((L, PAIR), jnp.float32),
        pltpu.VMEM((NBLK * VEC * L,), jnp.float32),
        pltpu.VMEM((PIECE, PAIR), jnp.float32),
        pltpu.VMEM((PIECE, PAIR), jnp.float32),
        pltpu.VMEM((RPW,), jnp.float32),
        pltpu.SemaphoreType.DMA,
        pltpu.SemaphoreType.DMA,
        pltpu.SemaphoreType.DMA,
    ],
    compiler_params=pltpu.CompilerParams(needs_layout_passes=False),
)
def _sc_cosine(wt_hbm, et_hbm, wid_hbm, eid_hbm, out_hbm,
               ids_v, perm_v, hv_v, ents_v, erow_v, E_v, buf0, buf1, out_v,
               sem0, sem1, sem_m):
    wid = lax.axis_index("s") * NC + lax.axis_index("c")
    iota = lax.iota(jnp.int32, L)
    zf = jnp.zeros((L,), jnp.float32)

    pltpu.sync_copy(wid_hbm.at[pl.ds(wid * RPW, RPW)], ids_v)
    for t in range(NBLK):
        pltpu.sync_copy(eid_hbm.at[pl.ds(wid * EPW + t * L, L)], ents_v.at[t])

    def _perm(j2, carry):
        t = j2 // JPB
        j = j2 - t * JPB
        v = plsc.load_gather(ids_v, [iota * JPB + (t * RPB + j)])
        row = jnp.zeros((L,), jnp.int32) + (j2 // JPP)
        col = (j2 % JPP) * L + iota
        plsc.store_scatter(perm_v, [row, col], lax.shift_right_logical(v, 1))
        hv_v[pl.ds(j2 * L, L)] = lax.shift_left(v & 1, 6)
        return carry

    lax.fori_loop(0, NBLK * JPB, _perm, 0)

    for t in range(NBLK):
        ev = ents_v[t]
        epair = lax.shift_right_logical(ev, 1)
        eh = lax.shift_left(ev & 1, 6)
        pltpu.async_copy(et_hbm.at[epair], erow_v, sem_m).wait()

        def _esq(k, sq):
            g = plsc.load_gather(erow_v, [iota, eh + k])
            return sq + g * g

        sq = lax.fori_loop(0, VEC, _esq, zf)
        rs = _rsqrt(jnp.maximum(sq, 1e-30))
        n = sq * rs
        scale = jnp.where(sq > 1.0, 1.0 / (n + 1e-7), jnp.ones((L,), jnp.float32))

        def _escale(k, carry):
            g = plsc.load_gather(erow_v, [iota, eh + k])
            E_v[pl.ds(t * VEC * L + k * L, L)] = g * carry
            return carry

        lax.fori_loop(0, VEC, _escale, scale)

    bufs = (buf0, buf1)
    sems = (sem0, sem1)
    for b in range(2):
        pltpu.async_copy(wt_hbm.at[perm_v.at[b]], bufs[b], sems[b])

    def _compute_piece(m, buf):
        t = m // PPB
        mm = m - t * PPB

        def _jbody(jj, carry):
            j = mm * JPP + jj
            h64 = hv_v[pl.ds(m * PIECE + jj * L, L)]
            row = iota + jj * L

            def _kbody(k, acc):
                sq, dot = acc
                v = plsc.load_gather(buf, [row, h64 + k])
                e = E_v[pl.ds(t * VEC * L + k * L, L)]
                return (sq + v * v, dot + v * e)

            sq, dot = (zf, zf)
            for k in range(VEC):
                sq, dot = _kbody(k, (sq, dot))
            val = dot * _rsqrt(jnp.maximum(sq, 1e-30))
            plsc.store_scatter(out_v, [iota * JPB + (t * RPB + j)], val)
            return carry

        lax.fori_loop(0, JPP, _jbody, 0)

    def _main(m2, carry):
        for b in range(2):
            m = m2 * 2 + b
            pltpu.make_async_copy(wt_hbm.at[perm_v.at[m]], bufs[b],
                                  sems[b]).wait()
            _compute_piece(m, bufs[b])
            nxt = m + 2

            @pl.when(nxt < PPW)
            def _fire():
                pltpu.async_copy(wt_hbm.at[perm_v.at[nxt]], bufs[b], sems[b])

        return carry

    lax.fori_loop(0, PPW // 2, _main, 0)

    pltpu.sync_copy(out_v, out_hbm.at[pl.ds(wid * RPW, RPW)])


def kernel(word_table, ent_table, word_ids, entity_ids):
    wt2 = word_table.reshape(WORD_VOCAB // 2, PAIR)
    et2 = ent_table.reshape(ENT_COUNT // 2, PAIR)
    out_flat = _sc_cosine(wt2, et2, word_ids, entity_ids)
    return out_flat.reshape(ROWS // 10, 10)


# speedup vs baseline: 1.4393x; 1.4393x over previous
import functools

import jax
import jax.numpy as jnp
from jax import lax
from jax.experimental import pallas as pl
from jax.experimental.pallas import tpu as pltpu
from jax.experimental.pallas import tpu_sc as plsc

ENT_COUNT = 1000000
WORD_VOCAB = 100000
VEC = 64
ROWS = 204800
JPB = 200
NC, NS, L = 2, 16, 16
NW = NC * NS
RPW = ROWS // NW
EPW = 32
NBLK = EPW // L
RPB = L * JPB
PIECE = 128
PPW = RPW // PIECE
PPB = PPW // NBLK
JPP = PIECE // L
EVL = VEC * L
ETOT = NW * NBLK * EVL


def _rsqrt(x):
    i = plsc.bitcast(x, jnp.int32)
    y = plsc.bitcast(jnp.int32(0x5F3759DF) - lax.shift_right_logical(i, 1),
                     jnp.float32)
    for _ in range(3):
        y = y * (1.5 - 0.5 * x * y * y)
    return y


@functools.partial(
    pl.kernel,
    out_type=jax.ShapeDtypeStruct((ETOT,), jnp.float32),
    mesh=plsc.VectorSubcoreMesh(core_axis_name="c", subcore_axis_name="s",
                                num_cores=NC, num_subcores=NS),
    scratch_types=[
        pltpu.VMEM((NBLK, L), jnp.int32),
        pltpu.VMEM((L, 8, VEC), jnp.float32),
        pltpu.VMEM((NBLK * EVL,), jnp.float32),
        pltpu.SemaphoreType.DMA,
    ],
    compiler_params=pltpu.CompilerParams(needs_layout_passes=False),
)
def _sc_entities(et_hbm, eid_hbm, E_hbm, ents_v, erow_v, E_v, sem_m):
    wid = lax.axis_index("s") * NC + lax.axis_index("c")
    iota = lax.iota(jnp.int32, L)
    zf = jnp.zeros((L,), jnp.float32)
    zi = jnp.zeros((L,), jnp.int32)

    for t in range(NBLK):
        pltpu.sync_copy(eid_hbm.at[pl.ds(wid * EPW + t * L, L)], ents_v.at[t])

    for t in range(NBLK):
        ev = ents_v[t]
        sub = ev & 7
        descs = []
        for i in range(L):
            s = jnp.sum(jnp.where(iota == i, ev, zi))
            base = pl.multiple_of(
                lax.shift_left(lax.shift_right_logical(s, 3), 3), 8)
            descs.append(
                pltpu.async_copy(et_hbm.at[pl.ds(base, 8)], erow_v.at[i],
                                 sem_m))
        for d in descs:
            d.wait()

        def _esq(k, sq):
            g = plsc.load_gather(erow_v, [iota, sub, zi + k])
            return sq + g * g

        sq = lax.fori_loop(0, VEC, _esq, zf)
        rs = _rsqrt(jnp.maximum(sq, 1e-30))
        n = sq * rs
        scale = jnp.where(sq > 1.0, 1.0 / (n + 1e-7),
                          jnp.ones((L,), jnp.float32))

        def _escale(k, carry):
            g = plsc.load_gather(erow_v, [iota, sub, zi + k])
            E_v[pl.ds(t * EVL + k * L, L)] = g * carry
            return carry

        lax.fori_loop(0, VEC, _escale, scale)

    pltpu.sync_copy(E_v, E_hbm.at[pl.ds(wid * NBLK * EVL, NBLK * EVL)])


@functools.partial(
    pl.kernel,
    out_type=jax.ShapeDtypeStruct((ROWS,), jnp.float32),
    mesh=plsc.VectorSubcoreMesh(core_axis_name="c", subcore_axis_name="s",
                                num_cores=NC, num_subcores=NS),
    scratch_types=[
        pltpu.VMEM((RPW,), jnp.int32),
        pltpu.VMEM((PPW, PIECE), jnp.int32),
        pltpu.VMEM((NBLK * EVL,), jnp.float32),
        pltpu.VMEM((PIECE, VEC), jnp.float32),
        pltpu.VMEM((PIECE, VEC), jnp.float32),
        pltpu.VMEM((RPW,), jnp.float32),
        pltpu.SemaphoreType.DMA,
        pltpu.SemaphoreType.DMA,
    ],
    compiler_params=pltpu.CompilerParams(needs_layout_passes=False,
                                         use_tc_tiling_on_sc=False),
)
def _sc_cosine(wt_hbm, wid_hbm, E_hbm, out_hbm,
               ids_v, perm_v, E_v, buf0, buf1, out_v, sem0, sem1):
    wid = lax.axis_index("s") * NC + lax.axis_index("c")
    iota = lax.iota(jnp.int32, L)
    zf = jnp.zeros((L,), jnp.float32)
    zi = jnp.zeros((L,), jnp.int32)

    pltpu.sync_copy(wid_hbm.at[pl.ds(wid * RPW, RPW)], ids_v)
    pltpu.sync_copy(E_hbm.at[pl.ds(wid * NBLK * EVL, NBLK * EVL)], E_v)

    def _perm(j2, carry):
        t = j2 // JPB
        j = j2 - t * JPB
        v = plsc.load_gather(ids_v, [iota * JPB + (t * RPB + j)])
        row = zi + (j2 // JPP)
        col = (j2 % JPP) * L + iota
        plsc.store_scatter(perm_v, [row, col], v)
        return carry

    lax.fori_loop(0, NBLK * JPB, _perm, 0)

    bufs = (buf0, buf1)
    sems = (sem0, sem1)
    for b in range(2):
        pltpu.async_copy(wt_hbm.at[perm_v.at[b]], bufs[b], sems[b])

    def _compute_piece(m, buf):
        t = m // PPB
        mm = m - t * PPB

        def _jbody(jj, carry):
            j = mm * JPP + jj
            row = iota + jj * L

            def _kbody(k, acc):
                sq, dot = acc
                v = plsc.load_gather(buf, [row, zi + k])
                e = E_v[pl.ds(t * EVL + k * L, L)]
                return (sq + v * v, dot + v * e)

            sq, dot = (zf, zf)
            for k in range(VEC):
                sq, dot = _kbody(k, (sq, dot))
            val = dot * _rsqrt(jnp.maximum(sq, 1e-30))
            plsc.store_scatter(out_v, [iota * JPB + (t * RPB + j)], val)
            return carry

        lax.fori_loop(0, JPP, _jbody, 0)

    def _main(m2, carry):
        for b in range(2):
            m = m2 * 2 + b
            pltpu.make_async_copy(wt_hbm.at[perm_v.at[m]], bufs[b],
                                  sems[b]).wait()
            _compute_piece(m, bufs[b])
            nxt = m + 2

            @pl.when(nxt < PPW)
            def _fire():
                pltpu.async_copy(wt_hbm.at[perm_v.at[nxt]], bufs[b], sems[b])

        return carry

    lax.fori_loop(0, PPW // 2, _main, 0)

    pltpu.sync_copy(out_v, out_hbm.at[pl.ds(wid * RPW, RPW)])


def kernel(word_table, ent_table, word_ids, entity_ids):
    e_tables = _sc_entities(ent_table, entity_ids)
    out_flat = _sc_cosine(word_table, word_ids, e_tables)
    return out_flat.reshape(ROWS // 10, 10)


# speedup vs baseline: 1.5360x; 1.0672x over previous
import functools

import jax
import jax.numpy as jnp
from jax import lax
from jax.experimental import pallas as pl
from jax.experimental.pallas import tpu as pltpu
from jax.experimental.pallas import tpu_sc as plsc

ENT_COUNT = 1000000
WORD_VOCAB = 100000
VEC = 64
ROWS = 204800
JPB = 200
NC, NS, L = 2, 16, 16
NW = NC * NS
RPW = ROWS // NW
EPW = 32
NBLK = EPW // L
RPB = L * JPB
PIECE = 128
PPW = RPW // PIECE
PPB = PPW // NBLK
JPP = PIECE // L
PAIR = 2 * VEC
EVL = VEC * L
ETOT = NW * NBLK * EVL
WCH = 512
NCH = WORD_VOCAB // WCH
WREM = WORD_VOCAB - NCH * WCH


def _rsqrt(x):
    i = plsc.bitcast(x, jnp.int32)
    y = plsc.bitcast(jnp.int32(0x5F3759DF) - lax.shift_right_logical(i, 1),
                     jnp.float32)
    for _ in range(3):
        y = y * (1.5 - 0.5 * x * y * y)
    return y


@functools.partial(
    pl.kernel,
    out_type=(jax.ShapeDtypeStruct((ETOT,), jnp.float32),
              jax.ShapeDtypeStruct((WORD_VOCAB // 2, PAIR), jnp.float32)),
    mesh=plsc.VectorSubcoreMesh(core_axis_name="c", subcore_axis_name="s",
                                num_cores=NC, num_subcores=NS),
    scratch_types=[
        pltpu.VMEM((NBLK, L), jnp.int32),
        pltpu.VMEM((L, 8, VEC), jnp.float32),
        pltpu.VMEM((NBLK * EVL,), jnp.float32),
        pltpu.VMEM((WCH, VEC), jnp.float32),
        pltpu.VMEM((WCH // 2, PAIR), jnp.float32),
        pltpu.SemaphoreType.DMA,
    ],
    compiler_params=pltpu.CompilerParams(needs_layout_passes=False),
)
def _sc_prep(et_hbm, eid_hbm, wt_hbm, E_hbm, wt2_hbm,
             ents_v, erow_v, E_v, wbuf_in, wbuf, sem_m):
    wid = lax.axis_index("s") * NC + lax.axis_index("c")
    iota = lax.iota(jnp.int32, L)
    zf = jnp.zeros((L,), jnp.float32)
    zi = jnp.zeros((L,), jnp.int32)

    for t in range(NBLK):
        pltpu.sync_copy(eid_hbm.at[pl.ds(wid * EPW + t * L, L)], ents_v.at[t])

    for t in range(NBLK):
        ev = ents_v[t]
        sub = ev & 7
        descs = []
        for i in range(L):
            s = jnp.sum(jnp.where(iota == i, ev, zi))
            base = pl.multiple_of(
                lax.shift_left(lax.shift_right_logical(s, 3), 3), 8)
            descs.append(
                pltpu.async_copy(et_hbm.at[pl.ds(base, 8)], erow_v.at[i],
                                 sem_m))
        for d in descs:
            d.wait()

        def _esq(k, sq):
            g = plsc.load_gather(erow_v, [iota, sub, (zi + k) ^ iota])
            return sq + g * g

        sq = lax.fori_loop(0, VEC, _esq, zf)
        rs = _rsqrt(jnp.maximum(sq, 1e-30))
        n = sq * rs
        scale = jnp.where(sq > 1.0, 1.0 / (n + 1e-7),
                          jnp.ones((L,), jnp.float32))

        def _escale(k, carry):
            g = plsc.load_gather(erow_v, [iota, sub, (zi + k) ^ iota])
            E_v[pl.ds(t * EVL + k * L, L)] = g * carry
            return carry

        lax.fori_loop(0, VEC, _escale, scale)

    pltpu.sync_copy(E_v, E_hbm.at[pl.ds(wid * NBLK * EVL, NBLK * EVL)])

    def _bounce(nrows):
        def body(q, carry):
            for c4 in range(VEC // L):
                v = wbuf_in[q, pl.ds(c4 * L, L)]
                wbuf[q >> 1, pl.ds(((q & 1) << 6) + c4 * L, L)] = v
            return carry

        lax.fori_loop(0, nrows, body, 0)

    def _depad(i, carry):
        c = wid + NW * i

        @pl.when(c < NCH)
        def _full():
            pltpu.sync_copy(wt_hbm.at[pl.ds(c * WCH, WCH)], wbuf_in)
            _bounce(WCH)
            pltpu.sync_copy(wbuf, wt2_hbm.at[pl.ds(c * (WCH // 2), WCH // 2)])

        return carry

    lax.fori_loop(0, (NCH + NW - 1) // NW, _depad, 0)

    @pl.when(wid == NCH % NW)
    def _tail():
        pltpu.sync_copy(wt_hbm.at[pl.ds(NCH * WCH, WREM)],
                        wbuf_in.at[pl.ds(0, WREM)])
        _bounce(WREM)
        pltpu.sync_copy(wbuf.at[pl.ds(0, WREM // 2)],
                        wt2_hbm.at[pl.ds(NCH * (WCH // 2), WREM // 2)])


@functools.partial(
    pl.kernel,
    out_type=jax.ShapeDtypeStruct((ROWS,), jnp.float32),
    mesh=plsc.VectorSubcoreMesh(core_axis_name="c", subcore_axis_name="s",
                                num_cores=NC, num_subcores=NS),
    scratch_types=[
        pltpu.VMEM((RPW,), jnp.int32),
        pltpu.VMEM((PPW, PIECE), jnp.int32),
        pltpu.VMEM((RPW,), jnp.int32),
        pltpu.VMEM((NBLK * EVL,), jnp.float32),
        pltpu.VMEM((PIECE, PAIR), jnp.float32),
        pltpu.VMEM((PIECE, PAIR), jnp.float32),
        pltpu.VMEM((RPW,), jnp.float32),
        pltpu.SemaphoreType.DMA,
        pltpu.SemaphoreType.DMA,
    ],
    compiler_params=pltpu.CompilerParams(needs_layout_passes=False),
)
def _sc_cosine(wt2_hbm, wid_hbm, E_hbm, out_hbm,
               ids_v, perm_v, hv_v, E_v, buf0, buf1, out_v, sem0, sem1):
    wid = lax.axis_index("s") * NC + lax.axis_index("c")
    iota = lax.iota(jnp.int32, L)
    zf = jnp.zeros((L,), jnp.float32)
    zi = jnp.zeros((L,), jnp.int32)

    pltpu.sync_copy(wid_hbm.at[pl.ds(wid * RPW, RPW)], ids_v)
    pltpu.sync_copy(E_hbm.at[pl.ds(wid * NBLK * EVL, NBLK * EVL)], E_v)

    def _perm(j2, carry):
        t = j2 // JPB
        j = j2 - t * JPB
        v = plsc.load_gather(ids_v, [iota * JPB + (t * RPB + j)])
        row = zi + (j2 // JPP)
        col = (j2 % JPP) * L + iota
        plsc.store_scatter(perm_v, [row, col], lax.shift_right_logical(v, 1))
        hv_v[pl.ds(j2 * L, L)] = lax.shift_left(v & 1, 6)
        return carry

    lax.fori_loop(0, NBLK * JPB, _perm, 0)

    bufs = (buf0, buf1)
    sems = (sem0, sem1)
    for b in range(2):
        pltpu.async_copy(wt2_hbm.at[perm_v.at[b]], bufs[b], sems[b])

    xor_cols = [iota ^ b for b in range(L)]

    def _compute_piece(m, buf):
        t = m // PPB
        mm = m - t * PPB

        def _jbody(jj, carry):
            j = mm * JPP + jj
            h64 = hv_v[pl.ds(m * PIECE + jj * L, L)]
            base = (iota + jj * L) * PAIR + h64
            bases = [base, base + 16, base + 32, base + 48]

            def _kbody(k, acc):
                sq, dot = acc
                idx = bases[k >> 4] + xor_cols[k & 15]
                v = plsc.load_gather(buf, [zi, idx])
                e = E_v[pl.ds(t * EVL + k * L, L)]
                return (sq + v * v, dot + v * e)

            sq, dot = (zf, zf)
            for k in range(VEC):
                sq, dot = _kbody(k, (sq, dot))
            val = dot * _rsqrt(jnp.maximum(sq, 1e-30))
            plsc.store_scatter(out_v, [iota * JPB + (t * RPB + j)], val)
            return carry

        lax.fori_loop(0, JPP, _jbody, 0)

    def _main(m2, carry):
        for b in range(2):
            m = m2 * 2 + b
            pltpu.make_async_copy(wt2_hbm.at[perm_v.at[m]], bufs[b],
                                  sems[b]).wait()
            _compute_piece(m, bufs[b])
            nxt = m + 2

            @pl.when(nxt < PPW)
            def _fire():
                pltpu.async_copy(wt2_hbm.at[perm_v.at[nxt]], bufs[b], sems[b])

        return carry

    lax.fori_loop(0, PPW // 2, _main, 0)

    pltpu.sync_copy(out_v, out_hbm.at[pl.ds(wid * RPW, RPW)])


def kernel(word_table, ent_table, word_ids, entity_ids):
    e_tables, wt2 = _sc_prep(ent_table, entity_ids, word_table)
    out_flat = _sc_cosine(wt2, word_ids, e_tables)
    return out_flat.reshape(ROWS // 10, 10)
